# baseline (speedup 1.0000x reference)
import functools

import jax
import jax.numpy as jnp
from jax import lax
from jax.experimental import pallas as pl
from jax.experimental.pallas import tpu as pltpu
from jax.experimental.pallas import tpu_sc as plsc

_N, _D = 128, 32768
_NC, _NS = 2, 16
_NW = _NC * _NS
_RPW = _N // _NW


def _sc_argmax_body(x_hbm, out_hbm, buf, accv, sem0, sem1):
    c = lax.axis_index("c")
    s = lax.axis_index("s")
    w = s * _NC + c
    row0 = w * _RPW
    sems = (sem0, sem1)
    lane = lax.iota(jnp.int32, 16)
    acc = jnp.zeros((16,), jnp.int32)
    cp = pltpu.async_copy(x_hbm.at[row0], buf.at[0], sems[0])
    for r in range(_RPW):
        b = r % 2
        if r + 1 < _RPW:
            nxt = pltpu.async_copy(
                x_hbm.at[row0 + r + 1], buf.at[(r + 1) % 2], sems[(r + 1) % 2]
            )
        cp.wait()

        def step(i, carry):
            vmax, vidx = carry
            v = buf[b, pl.ds(i * 16, 16)]
            upd = v > vmax
            vmax = jnp.where(upd, v, vmax)
            vidx = jnp.where(upd, jnp.full((16,), i, jnp.int32), vidx)
            return vmax, vidx

        vmax, vidx = lax.fori_loop(
            0,
            _D // 16,
            step,
            (jnp.full((16,), -jnp.inf, jnp.float32), jnp.zeros((16,), jnp.int32)),
            unroll=8,
        )
        m = jnp.max(vmax)
        cand = jnp.where(vmax == m, vidx * 16 + lane, jnp.iinfo(jnp.int32).max)
        g = jnp.min(cand)
        acc = jnp.where(lane == r, g, acc)
        if r + 1 < _RPW:
            cp = nxt
    accv[...] = acc
    pltpu.sync_copy(accv, out_hbm.at[w])


_sc_argmax = pl.kernel(
    _sc_argmax_body,
    out_type=jax.ShapeDtypeStruct((_NW, 16), jnp.int32),
    mesh=plsc.VectorSubcoreMesh(core_axis_name="c", subcore_axis_name="s"),
    compiler_params=pltpu.CompilerParams(needs_layout_passes=False),
    scratch_types=[
        pltpu.VMEM((2, _D), jnp.float32),
        pltpu.VMEM((16,), jnp.int32),
        pltpu.SemaphoreType.DMA,
        pltpu.SemaphoreType.DMA,
    ],
)


def kernel(logits):
    idx = _sc_argmax(logits)[:, :_RPW].reshape(_N)
    col = jnp.arange(_D, dtype=jnp.int32)[None, :]
    inf = jnp.float32(jnp.inf)
    return jnp.where(col == idx[:, None], inf, -inf)
